# baseline (speedup 1.0000x reference)
import jax
import jax.numpy as jnp
from jax import lax
from jax.experimental import pallas as pl
from jax.experimental.pallas import tpu as pltpu
from jax.experimental.pallas import tpu_sc as plsc

VOCAB = 100000
E = 100
B = 16384
C = 5
L = 16
NC, NS = 2, 16
NW = NC * NS
BPW = B // NW
PW = BPW * C
CB = 64
ITERS = BPW // CB
PPI = CB * C
NK = 6
TAIL = E - NK * L


def _body(tgt_idx, ctx_idx, tgt_tab, ctx_tab, out,
          tgt_idx_v, ctx_idx_v, tgt_rows_v, ctx_rows_v, out_v,
          sem_t0, sem_t1, sem_c0, sem_c1):
    wid = lax.axis_index("s") * NC + lax.axis_index("c")
    iota = lax.iota(jnp.int32, L)
    lane15 = (iota == L - 1)
    tail_keep = (iota >= L - TAIL)

    pltpu.sync_copy(tgt_idx.at[pl.ds(wid * BPW, BPW)], tgt_idx_v)
    pltpu.sync_copy(ctx_idx.at[pl.ds(wid * PW, PW)], ctx_idx_v)

    def fire_ctx(it1, buf, sem):
        @plsc.parallel_loop(0, PPI // L)
        def _issue(g):
            vec = ctx_idx_v[pl.ds(it1 * PPI + g * L, L)]
            for j in range(L):
                pltpu.async_copy(ctx_tab.at[pl.ds(vec[j], 1), :],
                                 ctx_rows_v.at[buf, pl.ds(g * L + j, 1), :],
                                 sem)

    def fire_tgt(it1, buf, sem):
        @plsc.parallel_loop(0, CB // L)
        def _issue(g):
            vec = tgt_idx_v[pl.ds(it1 * CB + g * L, L)]
            for j in range(L):
                pltpu.async_copy(tgt_tab.at[pl.ds(vec[j], 1), :],
                                 tgt_rows_v.at[buf, pl.ds(g * L + j, 1), :],
                                 sem)

    def drain(buf, sem_t, sem_c):
        pltpu.make_async_copy(tgt_tab.at[pl.ds(0, CB), :],
                              tgt_rows_v.at[buf], sem_t).wait()
        pltpu.make_async_copy(ctx_tab.at[pl.ds(0, PPI), :],
                              ctx_rows_v.at[buf], sem_c).wait()

    def compute(it, buf):
        @plsc.parallel_loop(0, CB, unroll=4)
        def _row(b):
            w = [tgt_rows_v[buf, b, pl.ds(k * L, L)] for k in range(NK)]
            w_tail = tgt_rows_v[buf, b, pl.ds(E - L, L)]
            for c in range(C):
                q = b * C + c
                prod = [w[k] * ctx_rows_v[buf, q, pl.ds(k * L, L)]
                        for k in range(NK)]
                prod.append(jnp.where(
                    tail_keep,
                    w_tail * ctx_rows_v[buf, q, pl.ds(E - L, L)], 0.0))
                while len(prod) > 1:
                    prod = [prod[i] + prod[i + 1]
                            for i in range(0, len(prod) - 1, 2)] + (
                        [prod[-1]] if len(prod) % 2 else [])
                s = plsc.cumsum(prod[0])
                gp = jnp.full((L,), it * PPI + q, dtype=jnp.int32)
                plsc.store_scatter(out_v, [gp], s, mask=lane15)

    fire_tgt(0, 0, sem_t0)
    fire_ctx(0, 0, sem_c0)

    @pl.loop(0, ITERS, step=2)
    def _it2(it):
        fire_tgt(it + 1, 1, sem_t1)
        fire_ctx(it + 1, 1, sem_c1)
        drain(0, sem_t0, sem_c0)
        compute(it, 0)

        @pl.when(it + 2 < ITERS)
        def _():
            fire_tgt(it + 2, 0, sem_t0)
            fire_ctx(it + 2, 0, sem_c0)

        drain(1, sem_t1, sem_c1)
        compute(it + 1, 1)

    pltpu.sync_copy(out_v, out.at[pl.ds(wid * PW, PW)])


def kernel(target, context, target_table, context_table):
    tgt_idx = target.reshape(B).astype(jnp.int32)
    ctx_idx = context.reshape(B * C).astype(jnp.int32)
    mesh = plsc.VectorSubcoreMesh(core_axis_name="c", subcore_axis_name="s",
                                  num_cores=NC, num_subcores=NS)
    run = pl.kernel(
        _body,
        out_type=jax.ShapeDtypeStruct((B * C,), jnp.float32),
        mesh=mesh,
        compiler_params=pltpu.CompilerParams(needs_layout_passes=False,
                                             use_tc_tiling_on_sc=True,
                                             disable_bounds_checks=True,
                                             disable_semaphore_checks=True),
        scratch_types=[
            pltpu.VMEM((BPW,), jnp.int32),
            pltpu.VMEM((PW,), jnp.int32),
            pltpu.VMEM((2, CB, E), jnp.float32),
            pltpu.VMEM((2, PPI, E), jnp.float32),
            pltpu.VMEM((PW,), jnp.float32),
            pltpu.SemaphoreType.DMA,
            pltpu.SemaphoreType.DMA,
            pltpu.SemaphoreType.DMA,
            pltpu.SemaphoreType.DMA,
        ],
    )
    out = run(tgt_idx, ctx_idx, target_table, context_table)
    return out.reshape(B, C)


# speedup vs baseline: 1.0464x; 1.0464x over previous
import jax
import jax.numpy as jnp
from jax import lax
from jax.experimental import pallas as pl
from jax.experimental.pallas import tpu as pltpu
from jax.experimental.pallas import tpu_sc as plsc

VOCAB = 100000
E = 100
B = 16384
C = 5
L = 16
NC, NS = 2, 16
NW = NC * NS
BPW = B // NW
PW = BPW * C
CB = 64
ITERS = BPW // CB
PPI = CB * C
NK = 6
TAIL = E - NK * L


def _body(tgt_idx, ctx_idx, tgt_tab, ctx_tab, out,
          tgt_idx_v, ctx_idx_v, tgt_rows_v, ctx_rows_v, out_v,
          sem_t0, sem_t1, sem_c0, sem_c1):
    wid = lax.axis_index("s") * NC + lax.axis_index("c")
    iota = lax.iota(jnp.int32, L)
    lane15 = (iota == L - 1)
    tail_keep = (iota >= L - TAIL)

    pltpu.sync_copy(tgt_idx.at[pl.ds(wid * BPW, BPW)], tgt_idx_v)
    pltpu.sync_copy(ctx_idx.at[pl.ds(wid * PW, PW)], ctx_idx_v)

    def fire_ctx(it1, buf, sem):
        @plsc.parallel_loop(0, PPI // L)
        def _issue(g):
            vec = ctx_idx_v[pl.ds(it1 * PPI + g * L, L)]
            for j in range(L):
                pltpu.async_copy(ctx_tab.at[pl.ds(vec[j], 1), :],
                                 ctx_rows_v.at[buf, pl.ds(g * L + j, 1), :],
                                 sem)

    def fire_tgt(it1, buf, sem):
        @plsc.parallel_loop(0, CB // L)
        def _issue(g):
            vec = tgt_idx_v[pl.ds(it1 * CB + g * L, L)]
            for j in range(L):
                pltpu.async_copy(tgt_tab.at[pl.ds(vec[j], 1), :],
                                 tgt_rows_v.at[buf, pl.ds(g * L + j, 1), :],
                                 sem)

    def drain(buf, sem_t, sem_c):
        pltpu.make_async_copy(tgt_tab.at[pl.ds(0, CB), :],
                              tgt_rows_v.at[buf], sem_t).wait()
        pltpu.make_async_copy(ctx_tab.at[pl.ds(0, PPI), :],
                              ctx_rows_v.at[buf], sem_c).wait()

    def compute(it, buf):
        @plsc.parallel_loop(0, CB, unroll=2)
        def _row(b):
            w = [tgt_rows_v[buf, b, pl.ds(k * L, L)] for k in range(NK)]
            w_tail = tgt_rows_v[buf, b, pl.ds(E - L, L)]
            for c in range(C):
                q = b * C + c
                prod = [w[k] * ctx_rows_v[buf, q, pl.ds(k * L, L)]
                        for k in range(NK)]
                prod.append(jnp.where(
                    tail_keep,
                    w_tail * ctx_rows_v[buf, q, pl.ds(E - L, L)], 0.0))
                while len(prod) > 1:
                    prod = [prod[i] + prod[i + 1]
                            for i in range(0, len(prod) - 1, 2)] + (
                        [prod[-1]] if len(prod) % 2 else [])
                s = plsc.cumsum(prod[0])
                gp = jnp.full((L,), it * PPI + q, dtype=jnp.int32)
                plsc.store_scatter(out_v, [gp], s, mask=lane15)

    fire_tgt(0, 0, sem_t0)
    fire_ctx(0, 0, sem_c0)

    @pl.loop(0, ITERS, step=2)
    def _it2(it):
        fire_tgt(it + 1, 1, sem_t1)
        fire_ctx(it + 1, 1, sem_c1)
        drain(0, sem_t0, sem_c0)
        compute(it, 0)

        @pl.when(it + 2 < ITERS)
        def _():
            fire_tgt(it + 2, 0, sem_t0)
            fire_ctx(it + 2, 0, sem_c0)

        drain(1, sem_t1, sem_c1)
        compute(it + 1, 1)

    pltpu.sync_copy(out_v, out.at[pl.ds(wid * PW, PW)])


def kernel(target, context, target_table, context_table):
    tgt_idx = target.reshape(B).astype(jnp.int32)
    ctx_idx = context.reshape(B * C).astype(jnp.int32)
    mesh = plsc.VectorSubcoreMesh(core_axis_name="c", subcore_axis_name="s",
                                  num_cores=NC, num_subcores=NS)
    run = pl.kernel(
        _body,
        out_type=jax.ShapeDtypeStruct((B * C,), jnp.float32),
        mesh=mesh,
        compiler_params=pltpu.CompilerParams(needs_layout_passes=False,
                                             use_tc_tiling_on_sc=True,
                                             disable_bounds_checks=True,
                                             disable_semaphore_checks=True),
        scratch_types=[
            pltpu.VMEM((BPW,), jnp.int32),
            pltpu.VMEM((PW,), jnp.int32),
            pltpu.VMEM((2, CB, E), jnp.float32),
            pltpu.VMEM((2, PPI, E), jnp.float32),
            pltpu.VMEM((PW,), jnp.float32),
            pltpu.SemaphoreType.DMA,
            pltpu.SemaphoreType.DMA,
            pltpu.SemaphoreType.DMA,
            pltpu.SemaphoreType.DMA,
        ],
    )
    out = run(tgt_idx, ctx_idx, target_table, context_table)
    return out.reshape(B, C)
